# Initial kernel scaffold; baseline (speedup 1.0000x reference)
import jax
import jax.numpy as jnp
from jax.experimental import pallas as pl


def kernel(x, edge_index, pos, W_rel1, b_rel1, W_root1, g1, be1, W_rel2, b_rel2, W_root2, g2, be2, W_cls, b_cls):
    raise NotImplementedError("write your pallas kernel here")


# speedup vs baseline: 1.1146x; 1.1146x over previous
import functools

import jax
import jax.numpy as jnp
from jax import lax
from jax.experimental import pallas as pl
from jax.experimental.pallas import tpu as pltpu
from jax.experimental.pallas import tpu_sc as plsc

N = 10000
E = 320000
D = 128
H = 512
C = 10

NC, NS, L = 2, 16, 16
NW = NC * NS
NPAD = 10240
EC = 2000
NCHUNK = E // EC
MB = EC + 32
POSPAD = NPAD + 16


def _make_segmax(F, RS, PASSES):
    def body(h_hbm, src_hbm, dst_hbm, pos_hbm, out_hbm,
             pos_v, srcc, dstc, srcm, ldstm, ewm, rows, acc, sem):
        wid = lax.axis_index("s") * NC + lax.axis_index("c")
        pltpu.sync_copy(pos_hbm, pos_v.at[pl.ds(0, N)])
        def pospad_body(t, _):
            pos_v[pl.ds(N + t * 16, 16)] = jnp.zeros((16,), jnp.float32)
            return 0
        lax.fori_loop(0, (POSPAD - N) // 16, pospad_body, 0)

        iota = lax.iota(jnp.int32, L)
        neg_inf = jnp.full((L,), -jnp.inf, jnp.float32)

        for p in range(PASSES):
            r = wid * PASSES + p
            lo = r * RS

            def init_body(t, _):
                acc[pl.ds(t * 16, 16)] = neg_inf
                return 0
            lax.fori_loop(0, (RS + 1) * F // 16, init_body, 0)

            def chunk_body(c, _):
                base_e = c * EC
                pltpu.sync_copy(src_hbm.at[pl.ds(base_e, EC)], srcc)
                pltpu.sync_copy(dst_hbm.at[pl.ds(base_e, EC)], dstc)

                def scan_body(t, off):
                    sv = srcc[pl.ds(t * 16, 16)]
                    dv = dstc[pl.ds(t * 16, 16)]
                    m = (dv >= lo) & (dv < lo + RS)
                    plsc.store_compressed(srcm.at[pl.ds(off, 16)], sv, mask=m)
                    plsc.store_compressed(ldstm.at[pl.ds(off, 16)], dv - lo, mask=m)
                    return off + jnp.sum(m.astype(jnp.int32))
                M = lax.fori_loop(0, EC // 16, scan_body, 0)

                zero16 = jnp.zeros((16,), jnp.int32)
                dummy16 = jnp.full((16,), RS, jnp.int32)
                srcm[pl.ds(M, 16)] = zero16
                srcm[pl.ds(M + 16, 16)] = zero16
                ldstm[pl.ds(M, 16)] = dummy16
                ldstm[pl.ds(M + 16, 16)] = dummy16
                NG = lax.shift_right_logical(M + 31, 5)

                def ew_body(t, _):
                    s16 = srcm[pl.ds(t * 16, 16)]
                    d16 = ldstm[pl.ds(t * 16, 16)] + lo
                    ps = plsc.load_gather(pos_v, [s16])
                    pd = plsc.load_gather(pos_v, [d16])
                    ewm[pl.ds(t * 16, 16)] = 1.0 / (ps - pd)
                    return 0
                lax.fori_loop(0, 2 * NG, ew_body, 0)

                def grp_body(g, _):
                    pltpu.async_copy(
                        h_hbm.at[srcm.at[pl.ds(g * 32, 32)]], rows, sem).wait()

                    def edge_body(k, _):
                        i = g * 32 + k
                        isp = jnp.full((L,), 0, jnp.int32) + i
                        dsp = plsc.load_gather(ldstm, [isp])
                        esp = plsc.load_gather(ewm, [isp])
                        ksp = jnp.full((L,), 0, jnp.int32) + k
                        base = dsp * F + iota
                        for j in range(F // 16):
                            col = iota + (j * 16)
                            idx = base + (j * 16)
                            rv = plsc.load_gather(rows, [ksp, col])
                            av = plsc.load_gather(acc, [idx])
                            plsc.store_scatter(acc, [idx],
                                               jnp.maximum(av, rv * esp))
                        return 0
                    lax.fori_loop(0, 32, edge_body, 0)
                    return 0
                lax.fori_loop(0, NG, grp_body, 0)
                return 0
            lax.fori_loop(0, NCHUNK, chunk_body, 0)

            def fin_body(t, _):
                v = acc[pl.ds(t * 16, 16)]
                acc[pl.ds(t * 16, 16)] = jnp.where(v == neg_inf, 0.0, v)
                return 0
            lax.fori_loop(0, RS * F // 16, fin_body, 0)
            pltpu.sync_copy(acc.at[pl.ds(0, RS * F)],
                            out_hbm.at[pl.ds(lo * F, RS * F)])

    mesh = plsc.VectorSubcoreMesh(core_axis_name="c", subcore_axis_name="s",
                                  num_cores=NC, num_subcores=NS)
    return pl.kernel(
        body,
        out_type=jax.ShapeDtypeStruct((NPAD * F,), jnp.float32),
        mesh=mesh,
        compiler_params=pltpu.CompilerParams(needs_layout_passes=False),
        scratch_types=[
            pltpu.VMEM((POSPAD,), jnp.float32),
            pltpu.VMEM((EC,), jnp.int32),
            pltpu.VMEM((EC,), jnp.int32),
            pltpu.VMEM((MB,), jnp.int32),
            pltpu.VMEM((MB,), jnp.int32),
            pltpu.VMEM((MB,), jnp.float32),
            pltpu.VMEM((32, F), jnp.float32),
            pltpu.VMEM(((RS + 1) * F,), jnp.float32),
            pltpu.SemaphoreType.DMA,
        ],
    )


_segmax1 = _make_segmax(D, NPAD // NW, 1)
_segmax2 = _make_segmax(H, NPAD // (2 * NW), 2)


def _dense1_body(agg_ref, x_ref, wr_ref, wo_ref, b_ref, g_ref, be_ref, o_ref):
    z = jnp.dot(agg_ref[...], wr_ref[...], preferred_element_type=jnp.float32)
    z += jnp.dot(x_ref[...], wo_ref[...], preferred_element_type=jnp.float32)
    z += b_ref[...]
    mu = jnp.mean(z, axis=-1, keepdims=True)
    var = jnp.mean((z - mu) ** 2, axis=-1, keepdims=True)
    z = (z - mu) * lax.rsqrt(var + 1e-5) * g_ref[...] + be_ref[...]
    o_ref[...] = jnp.maximum(z, 0.0)


def _dense2_body(agg_ref, h_ref, wr_ref, wo_ref, b_ref, g_ref, be_ref,
                 wc_ref, bc_ref, o_ref):
    z = jnp.dot(agg_ref[...], wr_ref[...], preferred_element_type=jnp.float32)
    z += jnp.dot(h_ref[...], wo_ref[...], preferred_element_type=jnp.float32)
    z += b_ref[...]
    mu = jnp.mean(z, axis=-1, keepdims=True)
    var = jnp.mean((z - mu) ** 2, axis=-1, keepdims=True)
    z = (z - mu) * lax.rsqrt(var + 1e-5) * g_ref[...] + be_ref[...]
    z = jnp.maximum(z, 0.0)
    o_ref[...] = jnp.dot(z, wc_ref[...],
                         preferred_element_type=jnp.float32) + bc_ref[...]


_BN = 400


def _dense1(agg, x, wr, wo, b, g, be):
    return pl.pallas_call(
        _dense1_body,
        grid=(N // _BN,),
        in_specs=[
            pl.BlockSpec((_BN, D), lambda i: (i, 0)),
            pl.BlockSpec((_BN, D), lambda i: (i, 0)),
            pl.BlockSpec((D, H), lambda i: (0, 0)),
            pl.BlockSpec((D, H), lambda i: (0, 0)),
            pl.BlockSpec((1, H), lambda i: (0, 0)),
            pl.BlockSpec((1, H), lambda i: (0, 0)),
            pl.BlockSpec((1, H), lambda i: (0, 0)),
        ],
        out_specs=pl.BlockSpec((_BN, H), lambda i: (i, 0)),
        out_shape=jax.ShapeDtypeStruct((N, H), jnp.float32),
    )(agg, x, wr, wo, b, g, be)


def _dense2(agg, h, wr, wo, b, g, be, wc, bc):
    return pl.pallas_call(
        _dense2_body,
        grid=(N // _BN,),
        in_specs=[
            pl.BlockSpec((_BN, H), lambda i: (i, 0)),
            pl.BlockSpec((_BN, H), lambda i: (i, 0)),
            pl.BlockSpec((H, H), lambda i: (0, 0)),
            pl.BlockSpec((H, H), lambda i: (0, 0)),
            pl.BlockSpec((1, H), lambda i: (0, 0)),
            pl.BlockSpec((1, H), lambda i: (0, 0)),
            pl.BlockSpec((1, H), lambda i: (0, 0)),
            pl.BlockSpec((H, 128), lambda i: (0, 0)),
            pl.BlockSpec((1, 128), lambda i: (0, 0)),
        ],
        out_specs=pl.BlockSpec((_BN, 128), lambda i: (i, 0)),
        out_shape=jax.ShapeDtypeStruct((N, 128), jnp.float32),
    )(agg, h, wr, wo, b, g, be, wc, bc)


def kernel(x, edge_index, pos, W_rel1, b_rel1, W_root1, g1, be1,
           W_rel2, b_rel2, W_root2, g2, be2, W_cls, b_cls):
    src = edge_index[0]
    dst = edge_index[1]

    agg1 = _segmax1(x, src, dst, pos).reshape(NPAD, D)[:N]
    h1 = _dense1(agg1, x, W_rel1, W_root1, b_rel1.reshape(1, H),
                 g1.reshape(1, H), be1.reshape(1, H))
    agg2 = _segmax2(h1, src, dst, pos).reshape(NPAD, H)[:N]
    wc = jnp.zeros((H, 128), jnp.float32).at[:, :C].set(W_cls)
    bc = jnp.zeros((1, 128), jnp.float32).at[0, :C].set(b_cls)
    out = _dense2(agg2, h1, W_rel2, W_root2, b_rel2.reshape(1, H),
                  g2.reshape(1, H), be2.reshape(1, H), wc, bc)
    return out[:, :C]


# speedup vs baseline: 2.8535x; 2.5601x over previous
import jax
import jax.numpy as jnp
from jax import lax
from jax.experimental import pallas as pl
from jax.experimental.pallas import tpu as pltpu
from jax.experimental.pallas import tpu_sc as plsc

N = 10000
E = 320000
D = 128
H = 512
C = 10

NC, NS, L = 2, 16, 16
NW = NC * NS
NPAD = 10240
EC = 2000
NCHUNK = E // EC
POSPAD = NPAD + 16


def _make_segmax(F, RS, PASSES, CAP, GX, SB):
    MB = CAP + 32
    NSUB = GX // SB

    def body(h_hbm, src_hbm, dst_hbm, pos_hbm, out_hbm,
             pos_v, eib, srcm, ldstm, ewm, offs_v, rows, acc, semE, semR):
        wid = lax.axis_index("s") * NC + lax.axis_index("c")
        pltpu.sync_copy(pos_hbm, pos_v.at[pl.ds(0, N)])

        def pospad_body(t, _):
            pos_v[pl.ds(N + t * 16, 16)] = jnp.zeros((16,), jnp.float32)
            return 0
        lax.fori_loop(0, (POSPAD - N) // 16, pospad_body, 0)

        iota = lax.iota(jnp.int32, L)
        neg_inf = jnp.full((L,), -jnp.inf, jnp.float32)

        def fire_eib(c):
            b = lax.rem(c, 2)
            pltpu.async_copy(src_hbm.at[pl.ds(c * EC, EC)],
                             eib.at[pl.ds(b * (2 * EC), EC)], semE.at[b, 0])
            pltpu.async_copy(dst_hbm.at[pl.ds(c * EC, EC)],
                             eib.at[pl.ds(b * (2 * EC) + EC, EC)], semE.at[b, 1])

        def wait_eib(b):
            pltpu.make_async_copy(src_hbm.at[pl.ds(0, EC)],
                                  eib.at[pl.ds(b * (2 * EC), EC)],
                                  semE.at[b, 0]).wait()
            pltpu.make_async_copy(dst_hbm.at[pl.ds(0, EC)],
                                  eib.at[pl.ds(b * (2 * EC) + EC, EC)],
                                  semE.at[b, 1]).wait()

        def fire_rows(g):
            b = lax.rem(g, 2)
            pltpu.async_copy(h_hbm.at[srcm.at[pl.ds(g * GX, GX)]],
                             rows.at[pl.ds(b * GX, GX)], semR.at[b])

        def wait_rows(b):
            pltpu.make_async_copy(h_hbm.at[srcm.at[pl.ds(0, GX)]],
                                  rows.at[pl.ds(b * GX, GX)], semR.at[b]).wait()

        for p in range(PASSES):
            r = wid * PASSES + p
            lo = r * RS

            def init_body(t, _):
                acc[pl.ds(t * 16, 16)] = neg_inf
                return 0
            lax.fori_loop(0, (RS + 1) * F // 16, init_body, 0)

            def flush(M):
                zero16 = jnp.zeros((16,), jnp.int32)
                dummy16 = jnp.full((16,), RS, jnp.int32)
                for q in range(GX // 16):
                    srcm[pl.ds(M + q * 16, 16)] = zero16
                    ldstm[pl.ds(M + q * 16, 16)] = dummy16
                NG = lax.shift_right_logical(M + GX - 1, 4 if GX == 16 else 5)

                def ew_body(t, _):
                    s16 = srcm[pl.ds(t * 16, 16)]
                    d16 = ldstm[pl.ds(t * 16, 16)] + lo
                    ps = plsc.load_gather(pos_v, [s16])
                    pd = plsc.load_gather(pos_v, [d16])
                    ewm[pl.ds(t * 16, 16)] = 1.0 / (ps - pd)
                    return 0
                lax.fori_loop(0, NG * (GX // 16), ew_body, 0)

                @pl.when(NG > 0)
                def _():
                    fire_rows(0)

                    def g_body(g, _):
                        b = lax.rem(g, 2)

                        @pl.when(g + 1 < NG)
                        def _():
                            fire_rows(g + 1)
                        wait_rows(b)
                        bro = jnp.full((L,), 0, jnp.int32) + b * GX

                        def sub_body(u, _):
                            i0 = g * GX + u * SB
                            for k in range(SB):
                                isp = jnp.full((L,), 0, jnp.int32) + (i0 + k)
                                dsp = plsc.load_gather(ldstm, [isp])
                                esp = plsc.load_gather(ewm, [isp])
                                ksp = bro + (u * SB + k)
                                base = dsp * F + iota
                                for j in range(F // 16):
                                    col = iota + (j * 16)
                                    idx = base + (j * 16)
                                    rv = plsc.load_gather(rows, [ksp, col])
                                    av = plsc.load_gather(acc, [idx])
                                    plsc.store_scatter(
                                        acc, [idx], jnp.maximum(av, rv * esp))
                            return 0
                        lax.fori_loop(0, NSUB, sub_body, 0)
                        return 0
                    lax.fori_loop(0, NG, g_body, 0)

            fire_eib(0)

            def chunk_body(c, OFF):
                b = lax.rem(c, 2)

                @pl.when(c + 1 < NCHUNK)
                def _():
                    fire_eib(c + 1)
                wait_eib(b)

                def do_flush(off):
                    flush(off)
                    return 0

                OFF2 = lax.cond(OFF + EC > CAP, do_flush, lambda off: off, OFF)

                OFFsp = jnp.full((L,), 0, jnp.int32) + OFF2
                bofs = jnp.full((L,), 0, jnp.int32) + b * (2 * EC)
                lane0 = iota < 1

                def scan_body(t, run):
                    ecol = bofs + t * 16 + iota
                    sv = plsc.load_gather(eib, [ecol])
                    dv = plsc.load_gather(eib, [ecol + EC])
                    m = (dv >= lo) & (dv < lo + RS)
                    cnt = plsc.all_reduce_population_count(m)
                    plsc.store_compressed(srcm.at[pl.ds(OFF2 + t * 16, 16)],
                                          sv, mask=m)
                    plsc.store_compressed(ldstm.at[pl.ds(OFF2 + t * 16, 16)],
                                          dv - lo, mask=m)
                    plsc.store_scatter(offs_v, [jnp.full((L,), 0, jnp.int32) + t],
                                       OFFsp + run, mask=lane0)
                    plsc.store_scatter(offs_v, [jnp.full((L,), 0, jnp.int32) + (EC // 16 + t)],
                                       cnt, mask=lane0)
                    return run + cnt
                run_end = lax.fori_loop(0, EC // 16, scan_body,
                                        jnp.zeros((L,), jnp.int32))
                Mc = jnp.max(run_end)

                def move_body(t, _):
                    tsp = jnp.full((L,), 0, jnp.int32) + t
                    ot = plsc.load_gather(offs_v, [tsp])
                    ct = plsc.load_gather(offs_v, [tsp + EC // 16])
                    mk = iota < ct
                    gsl = jnp.full((L,), 0, jnp.int32) + (OFF2 + t * 16) + iota
                    sv = plsc.load_gather(srcm, [gsl])
                    dv = plsc.load_gather(ldstm, [gsl])
                    tgt = ot + iota
                    plsc.store_scatter(srcm, [tgt], sv, mask=mk)
                    plsc.store_scatter(ldstm, [tgt], dv, mask=mk)
                    return 0
                lax.fori_loop(0, EC // 16, move_body, 0)
                return OFF2 + Mc
            OFF_end = lax.fori_loop(0, NCHUNK, chunk_body, 0)
            flush(OFF_end)

            def fin_body(t, _):
                v = acc[pl.ds(t * 16, 16)]
                acc[pl.ds(t * 16, 16)] = jnp.where(v == neg_inf, 0.0, v)
                return 0
            lax.fori_loop(0, RS * F // 16, fin_body, 0)
            pltpu.sync_copy(acc.at[pl.ds(0, RS * F)],
                            out_hbm.at[pl.ds(lo * F, RS * F)])

    mesh = plsc.VectorSubcoreMesh(core_axis_name="c", subcore_axis_name="s",
                                  num_cores=NC, num_subcores=NS)
    return pl.kernel(
        body,
        out_type=jax.ShapeDtypeStruct((NPAD * F,), jnp.float32),
        mesh=mesh,
        compiler_params=pltpu.CompilerParams(needs_layout_passes=False),
        scratch_types=[
            pltpu.VMEM((POSPAD,), jnp.float32),
            pltpu.VMEM((4 * EC,), jnp.int32),
            pltpu.VMEM((MB,), jnp.int32),
            pltpu.VMEM((MB,), jnp.int32),
            pltpu.VMEM((MB,), jnp.float32),
            pltpu.VMEM((2 * (EC // 16),), jnp.int32),
            pltpu.VMEM((2 * GX, F), jnp.float32),
            pltpu.VMEM(((RS + 1) * F,), jnp.float32),
            pltpu.SemaphoreType.DMA((2, 2)),
            pltpu.SemaphoreType.DMA((2,)),
        ],
    )


_segmax1 = _make_segmax(D, NPAD // NW, 1, 4096, 32, 8)
_segmax2 = _make_segmax(H, NPAD // (2 * NW), 2, 3072, 16, 4)


def _dense1_body(agg_ref, x_ref, wr_ref, wo_ref, b_ref, g_ref, be_ref, o_ref):
    z = jnp.dot(agg_ref[...], wr_ref[...], preferred_element_type=jnp.float32)
    z += jnp.dot(x_ref[...], wo_ref[...], preferred_element_type=jnp.float32)
    z += b_ref[...]
    mu = jnp.mean(z, axis=-1, keepdims=True)
    var = jnp.mean((z - mu) ** 2, axis=-1, keepdims=True)
    z = (z - mu) * lax.rsqrt(var + 1e-5) * g_ref[...] + be_ref[...]
    o_ref[...] = jnp.maximum(z, 0.0)


def _dense2_body(agg_ref, h_ref, wr_ref, wo_ref, b_ref, g_ref, be_ref,
                 wc_ref, bc_ref, o_ref):
    z = jnp.dot(agg_ref[...], wr_ref[...], preferred_element_type=jnp.float32)
    z += jnp.dot(h_ref[...], wo_ref[...], preferred_element_type=jnp.float32)
    z += b_ref[...]
    mu = jnp.mean(z, axis=-1, keepdims=True)
    var = jnp.mean((z - mu) ** 2, axis=-1, keepdims=True)
    z = (z - mu) * lax.rsqrt(var + 1e-5) * g_ref[...] + be_ref[...]
    z = jnp.maximum(z, 0.0)
    o_ref[...] = jnp.dot(z, wc_ref[...],
                         preferred_element_type=jnp.float32) + bc_ref[...]


_BN = 400


def _dense1(agg, x, wr, wo, b, g, be):
    return pl.pallas_call(
        _dense1_body,
        grid=(N // _BN,),
        in_specs=[
            pl.BlockSpec((_BN, D), lambda i: (i, 0)),
            pl.BlockSpec((_BN, D), lambda i: (i, 0)),
            pl.BlockSpec((D, H), lambda i: (0, 0)),
            pl.BlockSpec((D, H), lambda i: (0, 0)),
            pl.BlockSpec((1, H), lambda i: (0, 0)),
            pl.BlockSpec((1, H), lambda i: (0, 0)),
            pl.BlockSpec((1, H), lambda i: (0, 0)),
        ],
        out_specs=pl.BlockSpec((_BN, H), lambda i: (i, 0)),
        out_shape=jax.ShapeDtypeStruct((N, H), jnp.float32),
    )(agg, x, wr, wo, b, g, be)


def _dense2(agg, h, wr, wo, b, g, be, wc, bc):
    return pl.pallas_call(
        _dense2_body,
        grid=(N // _BN,),
        in_specs=[
            pl.BlockSpec((_BN, H), lambda i: (i, 0)),
            pl.BlockSpec((_BN, H), lambda i: (i, 0)),
            pl.BlockSpec((H, H), lambda i: (0, 0)),
            pl.BlockSpec((H, H), lambda i: (0, 0)),
            pl.BlockSpec((1, H), lambda i: (0, 0)),
            pl.BlockSpec((1, H), lambda i: (0, 0)),
            pl.BlockSpec((1, H), lambda i: (0, 0)),
            pl.BlockSpec((H, 128), lambda i: (0, 0)),
            pl.BlockSpec((1, 128), lambda i: (0, 0)),
        ],
        out_specs=pl.BlockSpec((_BN, 128), lambda i: (i, 0)),
        out_shape=jax.ShapeDtypeStruct((N, 128), jnp.float32),
    )(agg, h, wr, wo, b, g, be, wc, bc)


def kernel(x, edge_index, pos, W_rel1, b_rel1, W_root1, g1, be1,
           W_rel2, b_rel2, W_root2, g2, be2, W_cls, b_cls):
    src = edge_index[0]
    dst = edge_index[1]
    agg1 = _segmax1(x, src, dst, pos).reshape(NPAD, D)[:N]
    h1 = _dense1(agg1, x, W_rel1, W_root1, b_rel1.reshape(1, H),
                 g1.reshape(1, H), be1.reshape(1, H))
    agg2 = _segmax2(h1, src, dst, pos).reshape(NPAD, H)[:N]
    wc = jnp.zeros((H, 128), jnp.float32).at[:, :C].set(W_cls)
    bc = jnp.zeros((1, 128), jnp.float32).at[0, :C].set(b_cls)
    out = _dense2(agg2, h1, W_rel2, W_root2, b_rel2.reshape(1, H),
                  g2.reshape(1, H), be2.reshape(1, H), wc, bc)
    return out[:, :C]


# speedup vs baseline: 4.3986x; 1.5415x over previous
import jax
import jax.numpy as jnp
from jax import lax
from jax.experimental import pallas as pl
from jax.experimental.pallas import tpu as pltpu
from jax.experimental.pallas import tpu_sc as plsc

N = 10000
E = 320000
D = 128
H = 512
C = 10

NC, NS, L = 2, 16, 16
NW = NC * NS
NPAD = 10240
EC = 2000
NCHUNK = E // EC
POSPAD = NPAD + 16


def _make_segmax(F, RS, PASSES, CAP, GX, SB):
    MB = CAP + 32
    NSUB = GX // SB

    def body(h_hbm, src_hbm, dst_hbm, pos_hbm, out_hbm,
             pos_v, eib, srcm, ldstm, ewm, offs_v, rows, acc, semE, semR):
        wid = lax.axis_index("s") * NC + lax.axis_index("c")
        pltpu.sync_copy(pos_hbm, pos_v.at[pl.ds(0, N)])

        def pospad_body(t, _):
            pos_v[pl.ds(N + t * 16, 16)] = jnp.zeros((16,), jnp.float32)
            return 0
        lax.fori_loop(0, (POSPAD - N) // 16, pospad_body, 0)

        iota = lax.iota(jnp.int32, L)
        neg_inf = jnp.full((L,), -jnp.inf, jnp.float32)

        def fire_eib(c):
            b = lax.rem(c, 2)
            pltpu.async_copy(src_hbm.at[pl.ds(c * EC, EC)],
                             eib.at[pl.ds(b * (2 * EC), EC)], semE.at[b, 0])
            pltpu.async_copy(dst_hbm.at[pl.ds(c * EC, EC)],
                             eib.at[pl.ds(b * (2 * EC) + EC, EC)], semE.at[b, 1])

        def wait_eib(b):
            pltpu.make_async_copy(src_hbm.at[pl.ds(0, EC)],
                                  eib.at[pl.ds(b * (2 * EC), EC)],
                                  semE.at[b, 0]).wait()
            pltpu.make_async_copy(dst_hbm.at[pl.ds(0, EC)],
                                  eib.at[pl.ds(b * (2 * EC) + EC, EC)],
                                  semE.at[b, 1]).wait()

        def fire_rows(g):
            b = lax.rem(g, 2)
            pltpu.async_copy(h_hbm.at[srcm.at[pl.ds(g * GX, GX)]],
                             rows.at[pl.ds(b * GX, GX)], semR.at[b])

        def wait_rows(b):
            pltpu.make_async_copy(h_hbm.at[srcm.at[pl.ds(0, GX)]],
                                  rows.at[pl.ds(b * GX, GX)], semR.at[b]).wait()

        for p in range(PASSES):
            r = wid * PASSES + p
            lo = r * RS

            def init_body(t, _):
                acc[pl.ds(t * 16, 16)] = neg_inf
                return 0
            lax.fori_loop(0, (RS + 1) * F // 16, init_body, 0)

            def flush(M):
                zero16 = jnp.zeros((16,), jnp.int32)
                dummy16 = jnp.full((16,), RS, jnp.int32)
                for q in range(GX // 16):
                    srcm[pl.ds(M + q * 16, 16)] = zero16
                    ldstm[pl.ds(M + q * 16, 16)] = dummy16
                NG = lax.shift_right_logical(M + GX - 1, 4 if GX == 16 else 5)

                def ew_body(t, _):
                    s16 = srcm[pl.ds(t * 16, 16)]
                    d16 = ldstm[pl.ds(t * 16, 16)] + lo
                    ps = plsc.load_gather(pos_v, [s16])
                    pd = plsc.load_gather(pos_v, [d16])
                    ewm[pl.ds(t * 16, 16)] = 1.0 / (ps - pd)
                    return 0
                lax.fori_loop(0, NG * (GX // 16), ew_body, 0)

                @pl.when(NG > 0)
                def _():
                    fire_rows(0)

                    def g_body(g, _):
                        b = lax.rem(g, 2)

                        @pl.when(g + 1 < NG)
                        def _():
                            fire_rows(g + 1)
                        wait_rows(b)
                        bro = jnp.full((L,), 0, jnp.int32) + b * GX

                        def sub_body(u, _):
                            i0 = g * GX + u * SB
                            for k in range(SB):
                                isp = jnp.full((L,), 0, jnp.int32) + (i0 + k)
                                dsp = plsc.load_gather(ldstm, [isp])
                                esp = plsc.load_gather(ewm, [isp])
                                ksp = bro + (u * SB + k)
                                base = dsp * F + iota
                                nj = F // 16
                                for j0 in range(0, nj, 16):
                                    jb = range(j0, min(j0 + 16, nj))
                                    avs = [plsc.load_gather(acc, [base + (j * 16)])
                                           for j in jb]
                                    mvs = [plsc.load_gather(
                                               rows, [ksp, iota + (j * 16)]) * esp
                                           for j in jb]
                                    for q, j in enumerate(jb):
                                        plsc.store_scatter(
                                            acc, [base + (j * 16)],
                                            jnp.maximum(avs[q], mvs[q]))
                            return 0
                        lax.fori_loop(0, NSUB, sub_body, 0)
                        return 0
                    lax.fori_loop(0, NG, g_body, 0)

            fire_eib(0)

            def chunk_body(c, OFF):
                b = lax.rem(c, 2)

                @pl.when(c + 1 < NCHUNK)
                def _():
                    fire_eib(c + 1)
                wait_eib(b)

                def do_flush(off):
                    flush(off)
                    return 0

                OFF2 = lax.cond(OFF + EC > CAP, do_flush, lambda off: off, OFF)

                OFFsp = jnp.full((L,), 0, jnp.int32) + OFF2
                bofs = jnp.full((L,), 0, jnp.int32) + b * (2 * EC)
                lane0 = iota < 1

                def scan_one(t, run):
                    ecol = bofs + t * 16 + iota
                    sv = plsc.load_gather(eib, [ecol])
                    dv = plsc.load_gather(eib, [ecol + EC])
                    m = (dv >= lo) & (dv < lo + RS)
                    cnt = plsc.all_reduce_population_count(m)
                    plsc.store_compressed(srcm.at[pl.ds(OFF2 + t * 16, 16)],
                                          sv, mask=m)
                    plsc.store_compressed(ldstm.at[pl.ds(OFF2 + t * 16, 16)],
                                          dv - lo, mask=m)
                    plsc.store_scatter(offs_v, [jnp.full((L,), 0, jnp.int32) + t],
                                       OFFsp + run, mask=lane0)
                    plsc.store_scatter(offs_v, [jnp.full((L,), 0, jnp.int32) + (EC // 16 + t)],
                                       cnt, mask=lane0)
                    return run + cnt

                def scan_body(tt, run):
                    run = scan_one(tt * 5, run)
                    run = scan_one(tt * 5 + 1, run)
                    run = scan_one(tt * 5 + 2, run)
                    run = scan_one(tt * 5 + 3, run)
                    run = scan_one(tt * 5 + 4, run)
                    return run
                run_end = lax.fori_loop(0, EC // 80, scan_body,
                                        jnp.zeros((L,), jnp.int32))
                Mc = jnp.max(run_end)

                def move_one(t):
                    tsp = jnp.full((L,), 0, jnp.int32) + t
                    ot = plsc.load_gather(offs_v, [tsp])
                    ct = plsc.load_gather(offs_v, [tsp + EC // 16])
                    mk = iota < ct
                    gsl = jnp.full((L,), 0, jnp.int32) + (OFF2 + t * 16) + iota
                    sv = plsc.load_gather(srcm, [gsl])
                    dv = plsc.load_gather(ldstm, [gsl])
                    tgt = ot + iota
                    plsc.store_scatter(srcm, [tgt], sv, mask=mk)
                    plsc.store_scatter(ldstm, [tgt], dv, mask=mk)

                def move_body(tt, _):
                    for q in range(5):
                        move_one(tt * 5 + q)
                    return 0
                lax.fori_loop(0, EC // 80, move_body, 0)
                return OFF2 + Mc
            OFF_end = lax.fori_loop(0, NCHUNK, chunk_body, 0)
            flush(OFF_end)

            def fin_body(t, _):
                v = acc[pl.ds(t * 16, 16)]
                acc[pl.ds(t * 16, 16)] = jnp.where(v == neg_inf, 0.0, v)
                return 0
            lax.fori_loop(0, RS * F // 16, fin_body, 0)
            pltpu.sync_copy(acc.at[pl.ds(0, RS * F)],
                            out_hbm.at[pl.ds(lo * F, RS * F)])

    mesh = plsc.VectorSubcoreMesh(core_axis_name="c", subcore_axis_name="s",
                                  num_cores=NC, num_subcores=NS)
    return pl.kernel(
        body,
        out_type=jax.ShapeDtypeStruct((NPAD * F,), jnp.float32),
        mesh=mesh,
        compiler_params=pltpu.CompilerParams(needs_layout_passes=False),
        scratch_types=[
            pltpu.VMEM((POSPAD,), jnp.float32),
            pltpu.VMEM((4 * EC,), jnp.int32),
            pltpu.VMEM((MB,), jnp.int32),
            pltpu.VMEM((MB,), jnp.int32),
            pltpu.VMEM((MB,), jnp.float32),
            pltpu.VMEM((2 * (EC // 16),), jnp.int32),
            pltpu.VMEM((2 * GX, F), jnp.float32),
            pltpu.VMEM(((RS + 1) * F,), jnp.float32),
            pltpu.SemaphoreType.DMA((2, 2)),
            pltpu.SemaphoreType.DMA((2,)),
        ],
    )


_segmax1 = _make_segmax(D, NPAD // NW, 1, 4096, 32, 8)
_segmax2 = _make_segmax(H, NPAD // (2 * NW), 2, 3072, 16, 4)


def _dense1_body(agg_ref, x_ref, wr_ref, wo_ref, b_ref, g_ref, be_ref, o_ref):
    z = jnp.dot(agg_ref[...], wr_ref[...], preferred_element_type=jnp.float32)
    z += jnp.dot(x_ref[...], wo_ref[...], preferred_element_type=jnp.float32)
    z += b_ref[...]
    mu = jnp.mean(z, axis=-1, keepdims=True)
    var = jnp.mean((z - mu) ** 2, axis=-1, keepdims=True)
    z = (z - mu) * lax.rsqrt(var + 1e-5) * g_ref[...] + be_ref[...]
    o_ref[...] = jnp.maximum(z, 0.0)


def _dense2_body(agg_ref, h_ref, wr_ref, wo_ref, b_ref, g_ref, be_ref,
                 wc_ref, bc_ref, o_ref):
    z = jnp.dot(agg_ref[...], wr_ref[...], preferred_element_type=jnp.float32)
    z += jnp.dot(h_ref[...], wo_ref[...], preferred_element_type=jnp.float32)
    z += b_ref[...]
    mu = jnp.mean(z, axis=-1, keepdims=True)
    var = jnp.mean((z - mu) ** 2, axis=-1, keepdims=True)
    z = (z - mu) * lax.rsqrt(var + 1e-5) * g_ref[...] + be_ref[...]
    z = jnp.maximum(z, 0.0)
    o_ref[...] = jnp.dot(z, wc_ref[...],
                         preferred_element_type=jnp.float32) + bc_ref[...]


_BN = 400


def _dense1(agg, x, wr, wo, b, g, be):
    return pl.pallas_call(
        _dense1_body,
        grid=(N // _BN,),
        in_specs=[
            pl.BlockSpec((_BN, D), lambda i: (i, 0)),
            pl.BlockSpec((_BN, D), lambda i: (i, 0)),
            pl.BlockSpec((D, H), lambda i: (0, 0)),
            pl.BlockSpec((D, H), lambda i: (0, 0)),
            pl.BlockSpec((1, H), lambda i: (0, 0)),
            pl.BlockSpec((1, H), lambda i: (0, 0)),
            pl.BlockSpec((1, H), lambda i: (0, 0)),
        ],
        out_specs=pl.BlockSpec((_BN, H), lambda i: (i, 0)),
        out_shape=jax.ShapeDtypeStruct((N, H), jnp.float32),
    )(agg, x, wr, wo, b, g, be)


def _dense2(agg, h, wr, wo, b, g, be, wc, bc):
    return pl.pallas_call(
        _dense2_body,
        grid=(N // _BN,),
        in_specs=[
            pl.BlockSpec((_BN, H), lambda i: (i, 0)),
            pl.BlockSpec((_BN, H), lambda i: (i, 0)),
            pl.BlockSpec((H, H), lambda i: (0, 0)),
            pl.BlockSpec((H, H), lambda i: (0, 0)),
            pl.BlockSpec((1, H), lambda i: (0, 0)),
            pl.BlockSpec((1, H), lambda i: (0, 0)),
            pl.BlockSpec((1, H), lambda i: (0, 0)),
            pl.BlockSpec((H, 128), lambda i: (0, 0)),
            pl.BlockSpec((1, 128), lambda i: (0, 0)),
        ],
        out_specs=pl.BlockSpec((_BN, 128), lambda i: (i, 0)),
        out_shape=jax.ShapeDtypeStruct((N, 128), jnp.float32),
    )(agg, h, wr, wo, b, g, be, wc, bc)


def kernel(x, edge_index, pos, W_rel1, b_rel1, W_root1, g1, be1,
           W_rel2, b_rel2, W_root2, g2, be2, W_cls, b_cls):
    src = edge_index[0]
    dst = edge_index[1]
    agg1 = _segmax1(x, src, dst, pos).reshape(NPAD, D)[:N]
    h1 = _dense1(agg1, x, W_rel1, W_root1, b_rel1.reshape(1, H),
                 g1.reshape(1, H), be1.reshape(1, H))
    agg2 = _segmax2(h1, src, dst, pos).reshape(NPAD, H)[:N]
    wc = jnp.zeros((H, 128), jnp.float32).at[:, :C].set(W_cls)
    bc = jnp.zeros((1, 128), jnp.float32).at[0, :C].set(b_cls)
    out = _dense2(agg2, h1, W_rel2, W_root2, b_rel2.reshape(1, H),
                  g2.reshape(1, H), be2.reshape(1, H), wc, bc)
    return out[:, :C]


# speedup vs baseline: 5.9009x; 1.3415x over previous
import jax
import jax.numpy as jnp
from jax import lax
from jax.experimental import pallas as pl
from jax.experimental.pallas import tpu as pltpu
from jax.experimental.pallas import tpu_sc as plsc

N = 10000
E = 320000
D = 128
H = 512
C = 10

NC, NS, L = 2, 16, 16
NW = NC * NS
NPAD = 10240
POSPAD = NPAD + 16
ESEG = 326720


def _make_segmax(F, RS, PASSES, CAP, GX, SB, ECL, UF, list_out, list_in):
    MB = CAP + 32
    NSUB = GX // SB
    NT = ECL // 16
    NSEM = 3 if list_in else 2

    def body(*refs):
        if list_in:
            (h_hbm, lsrc_hbm, lldst_hbm, lew_hbm, lcnt_hbm, out_hbm,
             eib, ewb, srcm, ldstm, ewm, offs_v, cstage, rows, acc,
             semE, semR) = refs
        elif list_out:
            (h_hbm, src_hbm, dst_hbm, pos_hbm,
             out_hbm, lsrc_hbm, lldst_hbm, lew_hbm, lcnt_hbm,
             pos_v, eib, srcm, ldstm, ewm, offs_v, cstage, rows, acc,
             semE, semR) = refs
        else:
            raise AssertionError
        wid = lax.axis_index("s") * NC + lax.axis_index("c")
        iota = lax.iota(jnp.int32, L)
        neg_inf = jnp.full((L,), -jnp.inf, jnp.float32)

        if list_out:
            pltpu.sync_copy(pos_hbm, pos_v.at[pl.ds(0, N)])

            def pospad_body(t, _):
                pos_v[pl.ds(N + t * 16, 16)] = jnp.zeros((16,), jnp.float32)
                return 0
            lax.fori_loop(0, (POSPAD - N) // 16, pospad_body, 0)
            NCH = E // ECL
        else:
            pltpu.sync_copy(lcnt_hbm.at[pl.ds(wid * 16, 16)], cstage)
            cnt_all = jnp.max(cstage[pl.ds(0, 16)])
            NCH = lax.shift_right_logical(cnt_all + (ECL - 1), 11)

        lbase = wid * ESEG

        def fire_eib(c):
            b = lax.rem(c, 2)
            if list_in:
                o = lbase + c * ECL
                pltpu.async_copy(lsrc_hbm.at[pl.ds(o, ECL)],
                                 eib.at[pl.ds(b * (2 * ECL), ECL)], semE.at[b, 0])
                pltpu.async_copy(lldst_hbm.at[pl.ds(o, ECL)],
                                 eib.at[pl.ds(b * (2 * ECL) + ECL, ECL)],
                                 semE.at[b, 1])
                pltpu.async_copy(lew_hbm.at[pl.ds(o, ECL)],
                                 ewb.at[pl.ds(b * ECL, ECL)], semE.at[b, 2])
            else:
                pltpu.async_copy(src_hbm.at[pl.ds(c * ECL, ECL)],
                                 eib.at[pl.ds(b * (2 * ECL), ECL)], semE.at[b, 0])
                pltpu.async_copy(dst_hbm.at[pl.ds(c * ECL, ECL)],
                                 eib.at[pl.ds(b * (2 * ECL) + ECL, ECL)],
                                 semE.at[b, 1])

        def wait_eib(b):
            if list_in:
                pltpu.make_async_copy(lsrc_hbm.at[pl.ds(0, ECL)],
                                      eib.at[pl.ds(b * (2 * ECL), ECL)],
                                      semE.at[b, 0]).wait()
                pltpu.make_async_copy(lldst_hbm.at[pl.ds(0, ECL)],
                                      eib.at[pl.ds(b * (2 * ECL) + ECL, ECL)],
                                      semE.at[b, 1]).wait()
                pltpu.make_async_copy(lew_hbm.at[pl.ds(0, ECL)],
                                      ewb.at[pl.ds(b * ECL, ECL)],
                                      semE.at[b, 2]).wait()
            else:
                pltpu.make_async_copy(src_hbm.at[pl.ds(0, ECL)],
                                      eib.at[pl.ds(b * (2 * ECL), ECL)],
                                      semE.at[b, 0]).wait()
                pltpu.make_async_copy(dst_hbm.at[pl.ds(0, ECL)],
                                      eib.at[pl.ds(b * (2 * ECL) + ECL, ECL)],
                                      semE.at[b, 1]).wait()

        def fire_rows(g):
            b = lax.rem(g, 2)
            pltpu.async_copy(h_hbm.at[srcm.at[pl.ds(g * GX, GX)]],
                             rows.at[pl.ds(b * GX, GX)], semR.at[b])

        def wait_rows(b):
            pltpu.make_async_copy(h_hbm.at[srcm.at[pl.ds(0, GX)]],
                                  rows.at[pl.ds(b * GX, GX)], semR.at[b]).wait()

        for p in range(PASSES):
            r = wid * PASSES + p
            lo = r * RS
            plo = p * RS

            def init_body(t, _):
                acc[pl.ds(t * 16, 16)] = neg_inf
                return 0
            lax.fori_loop(0, (RS + 1) * F // 16, init_body, 0)

            def flush(M, W):
                zero16 = jnp.zeros((16,), jnp.int32)
                dummy16 = jnp.full((16,), RS, jnp.int32)
                for q in range(GX // 16):
                    srcm[pl.ds(M + q * 16, 16)] = zero16
                    ldstm[pl.ds(M + q * 16, 16)] = dummy16
                    if list_in:
                        ewm[pl.ds(M + q * 16, 16)] = jnp.zeros((16,), jnp.float32)
                NG = lax.shift_right_logical(M + GX - 1, 4 if GX == 16 else 5)

                if list_out:
                    def ew_body(t, _):
                        s16 = srcm[pl.ds(t * 16, 16)]
                        d16 = ldstm[pl.ds(t * 16, 16)] + lo
                        ps = plsc.load_gather(pos_v, [s16])
                        pd = plsc.load_gather(pos_v, [d16])
                        ewm[pl.ds(t * 16, 16)] = 1.0 / (ps - pd)
                        return 0
                    lax.fori_loop(0, NG * (GX // 16), ew_body, 0)
                    Wo = pl.multiple_of(lbase + W, 16)
                    pltpu.sync_copy(srcm.at[pl.ds(0, MB)],
                                    lsrc_hbm.at[pl.ds(Wo, MB)])
                    pltpu.sync_copy(ldstm.at[pl.ds(0, MB)],
                                    lldst_hbm.at[pl.ds(Wo, MB)])
                    pltpu.sync_copy(ewm.at[pl.ds(0, MB)],
                                    lew_hbm.at[pl.ds(Wo, MB)])

                @pl.when(NG > 0)
                def _():
                    fire_rows(0)

                    def g_body(g, _):
                        b = lax.rem(g, 2)

                        @pl.when(g + 1 < NG)
                        def _():
                            fire_rows(g + 1)
                        wait_rows(b)
                        bro = jnp.full((L,), 0, jnp.int32) + b * GX

                        def sub_body(u, _):
                            i0 = g * GX + u * SB
                            for k in range(SB):
                                isp = jnp.full((L,), 0, jnp.int32) + (i0 + k)
                                dsp = plsc.load_gather(ldstm, [isp])
                                esp = plsc.load_gather(ewm, [isp])
                                ksp = bro + (u * SB + k)
                                base = dsp * F + iota
                                nj = F // 16
                                for j0 in range(0, nj, 16):
                                    jb = range(j0, min(j0 + 16, nj))
                                    avs = [plsc.load_gather(acc, [base + (j * 16)])
                                           for j in jb]
                                    mvs = [plsc.load_gather(
                                               rows, [ksp, iota + (j * 16)]) * esp
                                           for j in jb]
                                    for q, j in enumerate(jb):
                                        plsc.store_scatter(
                                            acc, [base + (j * 16)],
                                            jnp.maximum(avs[q], mvs[q]))
                            return 0
                        lax.fori_loop(0, NSUB, sub_body, 0)
                        return 0
                    lax.fori_loop(0, NG, g_body, 0)

            if not list_in:
                fire_eib(0)
            else:
                @pl.when(NCH > 0)
                def _():
                    fire_eib(0)

            def chunk_body(c, carry):
                OFF, W = carry
                b = lax.rem(c, 2)

                @pl.when(c + 1 < NCH)
                def _():
                    fire_eib(c + 1)
                wait_eib(b)

                def do_flush(ow):
                    off, w = ow
                    flush(off, w)
                    return (0, w + lax.bitwise_and(off + 15, -16))

                OFF2, W2 = lax.cond(OFF + ECL > CAP, do_flush,
                                    lambda ow: ow, (OFF, W))

                OFFsp = jnp.full((L,), 0, jnp.int32) + OFF2
                bofs = jnp.full((L,), 0, jnp.int32) + b * (2 * ECL)
                if list_in:
                    bofs_e = jnp.full((L,), 0, jnp.int32) + b * ECL
                    cbase = jnp.full((L,), 0, jnp.int32) + c * ECL
                    csp = jnp.full((L,), 0, jnp.int32) + cnt_all
                lane0 = iota < 1

                def scan_one(t, run):
                    ecol = bofs + t * 16 + iota
                    sv = plsc.load_gather(eib, [ecol])
                    dv = plsc.load_gather(eib, [ecol + ECL])
                    if list_in:
                        m = ((dv >= plo) & (dv < plo + RS)
                             & (cbase + (t * 16) + iota < csp))
                        ev = plsc.load_gather(ewb, [bofs_e + t * 16 + iota])
                        dloc = dv - plo
                    else:
                        m = (dv >= lo) & (dv < lo + RS)
                        dloc = dv - lo
                    cnt = plsc.all_reduce_population_count(m)
                    plsc.store_compressed(srcm.at[pl.ds(OFF2 + t * 16, 16)],
                                          sv, mask=m)
                    plsc.store_compressed(ldstm.at[pl.ds(OFF2 + t * 16, 16)],
                                          dloc, mask=m)
                    if list_in:
                        plsc.store_compressed(ewm.at[pl.ds(OFF2 + t * 16, 16)],
                                              ev, mask=m)
                    plsc.store_scatter(offs_v, [jnp.full((L,), 0, jnp.int32) + t],
                                       OFFsp + run, mask=lane0)
                    plsc.store_scatter(offs_v,
                                       [jnp.full((L,), 0, jnp.int32) + (NT + t)],
                                       cnt, mask=lane0)
                    return run + cnt

                def scan_body(tt, run):
                    for q in range(UF):
                        run = scan_one(tt * UF + q, run)
                    return run
                run_end = lax.fori_loop(0, NT // UF, scan_body,
                                        jnp.zeros((L,), jnp.int32))
                Mc = jnp.max(run_end)

                def move_one(t):
                    tsp = jnp.full((L,), 0, jnp.int32) + t
                    ot = plsc.load_gather(offs_v, [tsp])
                    ct = plsc.load_gather(offs_v, [tsp + NT])
                    mk = iota < ct
                    gsl = jnp.full((L,), 0, jnp.int32) + (OFF2 + t * 16) + iota
                    sv = plsc.load_gather(srcm, [gsl])
                    dv = plsc.load_gather(ldstm, [gsl])
                    tgt = ot + iota
                    plsc.store_scatter(srcm, [tgt], sv, mask=mk)
                    plsc.store_scatter(ldstm, [tgt], dv, mask=mk)
                    if list_in:
                        ev = plsc.load_gather(ewm, [gsl])
                        plsc.store_scatter(ewm, [tgt], ev, mask=mk)

                def move_body(tt, _):
                    for q in range(UF):
                        move_one(tt * UF + q)
                    return 0
                lax.fori_loop(0, NT // UF, move_body, 0)
                return (OFF2 + Mc, W2)
            OFF_end, W_end = lax.fori_loop(0, NCH, chunk_body, (0, 0))
            flush(OFF_end, W_end)

            if list_out:
                Wfin = W_end + lax.bitwise_and(OFF_end + 15, -16)
                cstage[pl.ds(0, 16)] = jnp.full((L,), 0, jnp.int32) + Wfin
                pltpu.sync_copy(cstage, lcnt_hbm.at[pl.ds(wid * 16, 16)])

            def fin_body(t, _):
                v = acc[pl.ds(t * 16, 16)]
                acc[pl.ds(t * 16, 16)] = jnp.where(v == neg_inf, 0.0, v)
                return 0
            lax.fori_loop(0, RS * F // 16, fin_body, 0)
            pltpu.sync_copy(acc.at[pl.ds(0, RS * F)],
                            out_hbm.at[pl.ds(lo * F, RS * F)])

    mesh = plsc.VectorSubcoreMesh(core_axis_name="c", subcore_axis_name="s",
                                  num_cores=NC, num_subcores=NS)
    if list_out:
        out_type = (jax.ShapeDtypeStruct((NPAD * F,), jnp.float32),
                    jax.ShapeDtypeStruct((NW * ESEG,), jnp.int32),
                    jax.ShapeDtypeStruct((NW * ESEG,), jnp.int32),
                    jax.ShapeDtypeStruct((NW * ESEG,), jnp.float32),
                    jax.ShapeDtypeStruct((NW * 16,), jnp.int32))
    else:
        out_type = jax.ShapeDtypeStruct((NPAD * F,), jnp.float32)
    scratch = []
    if list_out:
        scratch.append(pltpu.VMEM((POSPAD,), jnp.float32))
    scratch.append(pltpu.VMEM((4 * ECL,), jnp.int32))
    if list_in:
        scratch.insert(1, pltpu.VMEM((2 * ECL,), jnp.float32))
    scratch += [
        pltpu.VMEM((MB,), jnp.int32),
        pltpu.VMEM((MB,), jnp.int32),
        pltpu.VMEM((MB,), jnp.float32),
        pltpu.VMEM((2 * NT,), jnp.int32),
        pltpu.VMEM((16,), jnp.int32),
        pltpu.VMEM((2 * GX, F), jnp.float32),
        pltpu.VMEM(((RS + 1) * F,), jnp.float32),
        pltpu.SemaphoreType.DMA((2, NSEM)),
        pltpu.SemaphoreType.DMA((2,)),
    ]
    return pl.kernel(
        body,
        out_type=out_type,
        mesh=mesh,
        compiler_params=pltpu.CompilerParams(needs_layout_passes=False),
        scratch_types=scratch,
    )


_segmax1 = _make_segmax(D, NPAD // NW, 1, 4096, 32, 8, 2000, 5,
                        list_out=True, list_in=False)
_segmax2 = _make_segmax(H, NPAD // (2 * NW), 2, 3072, 16, 4, 2048, 4,
                        list_out=False, list_in=True)


def _dense1_body(agg_ref, x_ref, wr_ref, wo_ref, b_ref, g_ref, be_ref, o_ref):
    z = jnp.dot(agg_ref[...], wr_ref[...], preferred_element_type=jnp.float32)
    z += jnp.dot(x_ref[...], wo_ref[...], preferred_element_type=jnp.float32)
    z += b_ref[...]
    mu = jnp.mean(z, axis=-1, keepdims=True)
    var = jnp.mean((z - mu) ** 2, axis=-1, keepdims=True)
    z = (z - mu) * lax.rsqrt(var + 1e-5) * g_ref[...] + be_ref[...]
    o_ref[...] = jnp.maximum(z, 0.0)


def _dense2_body(agg_ref, h_ref, wr_ref, wo_ref, b_ref, g_ref, be_ref,
                 wc_ref, bc_ref, o_ref):
    z = jnp.dot(agg_ref[...], wr_ref[...], preferred_element_type=jnp.float32)
    z += jnp.dot(h_ref[...], wo_ref[...], preferred_element_type=jnp.float32)
    z += b_ref[...]
    mu = jnp.mean(z, axis=-1, keepdims=True)
    var = jnp.mean((z - mu) ** 2, axis=-1, keepdims=True)
    z = (z - mu) * lax.rsqrt(var + 1e-5) * g_ref[...] + be_ref[...]
    z = jnp.maximum(z, 0.0)
    o_ref[...] = jnp.dot(z, wc_ref[...],
                         preferred_element_type=jnp.float32) + bc_ref[...]


_BN = 400


def _dense1(agg, x, wr, wo, b, g, be):
    return pl.pallas_call(
        _dense1_body,
        grid=(N // _BN,),
        in_specs=[
            pl.BlockSpec((_BN, D), lambda i: (i, 0)),
            pl.BlockSpec((_BN, D), lambda i: (i, 0)),
            pl.BlockSpec((D, H), lambda i: (0, 0)),
            pl.BlockSpec((D, H), lambda i: (0, 0)),
            pl.BlockSpec((1, H), lambda i: (0, 0)),
            pl.BlockSpec((1, H), lambda i: (0, 0)),
            pl.BlockSpec((1, H), lambda i: (0, 0)),
        ],
        out_specs=pl.BlockSpec((_BN, H), lambda i: (i, 0)),
        out_shape=jax.ShapeDtypeStruct((N, H), jnp.float32),
    )(agg, x, wr, wo, b, g, be)


def _dense2(agg, h, wr, wo, b, g, be, wc, bc):
    return pl.pallas_call(
        _dense2_body,
        grid=(N // _BN,),
        in_specs=[
            pl.BlockSpec((_BN, H), lambda i: (i, 0)),
            pl.BlockSpec((_BN, H), lambda i: (i, 0)),
            pl.BlockSpec((H, H), lambda i: (0, 0)),
            pl.BlockSpec((H, H), lambda i: (0, 0)),
            pl.BlockSpec((1, H), lambda i: (0, 0)),
            pl.BlockSpec((1, H), lambda i: (0, 0)),
            pl.BlockSpec((1, H), lambda i: (0, 0)),
            pl.BlockSpec((H, 128), lambda i: (0, 0)),
            pl.BlockSpec((1, 128), lambda i: (0, 0)),
        ],
        out_specs=pl.BlockSpec((_BN, 128), lambda i: (i, 0)),
        out_shape=jax.ShapeDtypeStruct((N, 128), jnp.float32),
    )(agg, h, wr, wo, b, g, be, wc, bc)


def kernel(x, edge_index, pos, W_rel1, b_rel1, W_root1, g1, be1,
           W_rel2, b_rel2, W_root2, g2, be2, W_cls, b_cls):
    src = edge_index[0]
    dst = edge_index[1]
    agg1f, lsrc, lldst, lew, lcnt = _segmax1(x, src, dst, pos)
    agg1 = agg1f.reshape(NPAD, D)[:N]
    h1 = _dense1(agg1, x, W_rel1, W_root1, b_rel1.reshape(1, H),
                 g1.reshape(1, H), be1.reshape(1, H))
    agg2 = _segmax2(h1, lsrc, lldst, lew, lcnt).reshape(NPAD, H)[:N]
    wc = jnp.zeros((H, 128), jnp.float32).at[:, :C].set(W_cls)
    bc = jnp.zeros((1, 128), jnp.float32).at[0, :C].set(b_cls)
    out = _dense2(agg2, h1, W_rel2, W_root2, b_rel2.reshape(1, H),
                  g2.reshape(1, H), be2.reshape(1, H), wc, bc)
    return out[:, :C]


# speedup vs baseline: 6.0089x; 1.0183x over previous
import jax
import jax.numpy as jnp
from jax import lax
from jax.experimental import pallas as pl
from jax.experimental.pallas import tpu as pltpu
from jax.experimental.pallas import tpu_sc as plsc

N = 10000
E = 320000
D = 128
H = 512
C = 10

NC, NS, L = 2, 16, 16
NW = NC * NS
NPAD = 10240
POSPAD = NPAD + 16
ESEG = 330240


def _make_segmax(F, RS, PASSES, CAP, GX, SB, ECL, UF, list_out, list_in):
    MB = CAP + 32
    NSUB = GX // SB
    NT = ECL // 16
    NSEM = 3 if list_in else 2

    def body(*refs):
        if list_in:
            (h_hbm, lsrc_hbm, lldst_hbm, lew_hbm, lcnt_hbm, out_hbm,
             eib, ewb, srcm, ldstm, ewm, offs_v, cstage, rows, acc,
             semE, semR) = refs
        elif list_out:
            (h_hbm, src_hbm, dst_hbm, pos_hbm,
             out_hbm, lsrc_hbm, lldst_hbm, lew_hbm, lcnt_hbm,
             pos_v, eib, srcm, ldstm, ewm, offs_v, cstage, rows, acc,
             semE, semR) = refs
        else:
            raise AssertionError
        wid = lax.axis_index("s") * NC + lax.axis_index("c")
        iota = lax.iota(jnp.int32, L)
        neg_inf = jnp.full((L,), -jnp.inf, jnp.float32)

        if list_out:
            pltpu.sync_copy(pos_hbm, pos_v.at[pl.ds(0, N)])

            def pospad_body(t, _):
                pos_v[pl.ds(N + t * 16, 16)] = jnp.zeros((16,), jnp.float32)
                return 0
            lax.fori_loop(0, (POSPAD - N) // 16, pospad_body, 0)
            NCH = E // ECL
        else:
            pltpu.sync_copy(lcnt_hbm.at[pl.ds(wid * 16, 16)], cstage)
            cnt_all = jnp.max(cstage[pl.ds(0, 16)])
            NCH = lax.shift_right_logical(cnt_all + (ECL - 1), 11)

        lbase = wid * ESEG

        def fire_eib(c):
            b = lax.rem(c, 2)
            if list_in:
                o = lbase + c * ECL
                pltpu.async_copy(lsrc_hbm.at[pl.ds(o, ECL)],
                                 eib.at[pl.ds(b * (2 * ECL), ECL)], semE.at[b, 0])
                pltpu.async_copy(lldst_hbm.at[pl.ds(o, ECL)],
                                 eib.at[pl.ds(b * (2 * ECL) + ECL, ECL)],
                                 semE.at[b, 1])
                pltpu.async_copy(lew_hbm.at[pl.ds(o, ECL)],
                                 ewb.at[pl.ds(b * ECL, ECL)], semE.at[b, 2])
            else:
                pltpu.async_copy(src_hbm.at[pl.ds(c * ECL, ECL)],
                                 eib.at[pl.ds(b * (2 * ECL), ECL)], semE.at[b, 0])
                pltpu.async_copy(dst_hbm.at[pl.ds(c * ECL, ECL)],
                                 eib.at[pl.ds(b * (2 * ECL) + ECL, ECL)],
                                 semE.at[b, 1])

        def wait_eib(b):
            if list_in:
                pltpu.make_async_copy(lsrc_hbm.at[pl.ds(0, ECL)],
                                      eib.at[pl.ds(b * (2 * ECL), ECL)],
                                      semE.at[b, 0]).wait()
                pltpu.make_async_copy(lldst_hbm.at[pl.ds(0, ECL)],
                                      eib.at[pl.ds(b * (2 * ECL) + ECL, ECL)],
                                      semE.at[b, 1]).wait()
                pltpu.make_async_copy(lew_hbm.at[pl.ds(0, ECL)],
                                      ewb.at[pl.ds(b * ECL, ECL)],
                                      semE.at[b, 2]).wait()
            else:
                pltpu.make_async_copy(src_hbm.at[pl.ds(0, ECL)],
                                      eib.at[pl.ds(b * (2 * ECL), ECL)],
                                      semE.at[b, 0]).wait()
                pltpu.make_async_copy(dst_hbm.at[pl.ds(0, ECL)],
                                      eib.at[pl.ds(b * (2 * ECL) + ECL, ECL)],
                                      semE.at[b, 1]).wait()

        def fire_rows(g):
            b = lax.rem(g, 2)
            pltpu.async_copy(h_hbm.at[srcm.at[pl.ds(g * GX, GX)]],
                             rows.at[pl.ds(b * GX, GX)], semR.at[b])

        def wait_rows(b):
            pltpu.make_async_copy(h_hbm.at[srcm.at[pl.ds(0, GX)]],
                                  rows.at[pl.ds(b * GX, GX)], semR.at[b]).wait()

        for p in range(PASSES):
            r = wid * PASSES + p
            lo = r * RS
            plo = p * RS

            def init_body(t, _):
                acc[pl.ds(t * 16, 16)] = neg_inf
                return 0
            lax.fori_loop(0, (RS + 1) * F // 16, init_body, 0)

            def flush(M, W):
                zero16 = jnp.zeros((16,), jnp.int32)
                dummy16 = jnp.full((16,), RS, jnp.int32)
                for q in range(GX // 16):
                    srcm[pl.ds(M + q * 16, 16)] = zero16
                    ldstm[pl.ds(M + q * 16, 16)] = dummy16
                    if list_in:
                        ewm[pl.ds(M + q * 16, 16)] = jnp.zeros((16,), jnp.float32)
                NG = lax.shift_right_logical(M + GX - 1, 4 if GX == 16 else 5)

                if list_out:
                    def ew_body(t, _):
                        s16 = srcm[pl.ds(t * 16, 16)]
                        d16 = ldstm[pl.ds(t * 16, 16)] + lo
                        ps = plsc.load_gather(pos_v, [s16])
                        pd = plsc.load_gather(pos_v, [d16])
                        ewm[pl.ds(t * 16, 16)] = 1.0 / (ps - pd)
                        return 0
                    lax.fori_loop(0, NG * (GX // 16), ew_body, 0)
                    Wo = pl.multiple_of(lbase + W, 16)
                    pltpu.sync_copy(srcm.at[pl.ds(0, MB)],
                                    lsrc_hbm.at[pl.ds(Wo, MB)])
                    pltpu.sync_copy(ldstm.at[pl.ds(0, MB)],
                                    lldst_hbm.at[pl.ds(Wo, MB)])
                    pltpu.sync_copy(ewm.at[pl.ds(0, MB)],
                                    lew_hbm.at[pl.ds(Wo, MB)])

                @pl.when(NG > 0)
                def _():
                    fire_rows(0)

                    def g_body(g, _):
                        b = lax.rem(g, 2)

                        @pl.when(g + 1 < NG)
                        def _():
                            fire_rows(g + 1)
                        wait_rows(b)
                        bro = jnp.full((L,), 0, jnp.int32) + b * GX

                        def sub_body(u, _):
                            i0 = g * GX + u * SB
                            for k in range(SB):
                                isp = jnp.full((L,), 0, jnp.int32) + (i0 + k)
                                dsp = plsc.load_gather(ldstm, [isp])
                                esp = plsc.load_gather(ewm, [isp])
                                ksp = bro + (u * SB + k)
                                base = dsp * F + iota
                                nj = F // 16
                                for j0 in range(0, nj, 16):
                                    jb = range(j0, min(j0 + 16, nj))
                                    avs = [plsc.load_gather(acc, [base + (j * 16)])
                                           for j in jb]
                                    mvs = [plsc.load_gather(
                                               rows, [ksp, iota + (j * 16)]) * esp
                                           for j in jb]
                                    for q, j in enumerate(jb):
                                        plsc.store_scatter(
                                            acc, [base + (j * 16)],
                                            jnp.maximum(avs[q], mvs[q]))
                            return 0
                        lax.fori_loop(0, NSUB, sub_body, 0)
                        return 0
                    lax.fori_loop(0, NG, g_body, 0)

            if not list_in:
                fire_eib(0)
            else:
                @pl.when(NCH > 0)
                def _():
                    fire_eib(0)

            def chunk_body(c, carry):
                OFF, W = carry
                b = lax.rem(c, 2)

                @pl.when(c + 1 < NCH)
                def _():
                    fire_eib(c + 1)
                wait_eib(b)

                def do_flush(ow):
                    off, w = ow
                    flush(off, w)
                    return (0, w + lax.bitwise_and(off + 15, -16))

                OFF2, W2 = lax.cond(OFF + ECL > CAP, do_flush,
                                    lambda ow: ow, (OFF, W))

                OFFsp = jnp.full((L,), 0, jnp.int32) + OFF2
                bofs = jnp.full((L,), 0, jnp.int32) + b * (2 * ECL)
                if list_in:
                    bofs_e = jnp.full((L,), 0, jnp.int32) + b * ECL
                    cbase = jnp.full((L,), 0, jnp.int32) + c * ECL
                    csp = jnp.full((L,), 0, jnp.int32) + cnt_all
                lane0 = iota < 1

                def scan_one(t, run):
                    ecol = bofs + t * 16 + iota
                    sv = plsc.load_gather(eib, [ecol])
                    dv = plsc.load_gather(eib, [ecol + ECL])
                    if list_in:
                        m = ((dv >= plo) & (dv < plo + RS)
                             & (cbase + (t * 16) + iota < csp))
                        ev = plsc.load_gather(ewb, [bofs_e + t * 16 + iota])
                        dloc = dv - plo
                    else:
                        m = (dv >= lo) & (dv < lo + RS)
                        dloc = dv - lo
                    cnt = plsc.all_reduce_population_count(m)
                    plsc.store_compressed(srcm.at[pl.ds(OFF2 + t * 16, 16)],
                                          sv, mask=m)
                    plsc.store_compressed(ldstm.at[pl.ds(OFF2 + t * 16, 16)],
                                          dloc, mask=m)
                    if list_in:
                        plsc.store_compressed(ewm.at[pl.ds(OFF2 + t * 16, 16)],
                                              ev, mask=m)
                    plsc.store_scatter(offs_v, [jnp.full((L,), 0, jnp.int32) + t],
                                       OFFsp + run, mask=lane0)
                    plsc.store_scatter(offs_v,
                                       [jnp.full((L,), 0, jnp.int32) + (NT + t)],
                                       cnt, mask=lane0)
                    return run + cnt

                def scan_body(tt, run):
                    for q in range(UF):
                        run = scan_one(tt * UF + q, run)
                    return run
                run_end = lax.fori_loop(0, NT // UF, scan_body,
                                        jnp.zeros((L,), jnp.int32))
                Mc = jnp.max(run_end)

                def move_one(t):
                    tsp = jnp.full((L,), 0, jnp.int32) + t
                    ot = plsc.load_gather(offs_v, [tsp])
                    ct = plsc.load_gather(offs_v, [tsp + NT])
                    mk = iota < ct
                    gsl = jnp.full((L,), 0, jnp.int32) + (OFF2 + t * 16) + iota
                    sv = plsc.load_gather(srcm, [gsl])
                    dv = plsc.load_gather(ldstm, [gsl])
                    tgt = ot + iota
                    plsc.store_scatter(srcm, [tgt], sv, mask=mk)
                    plsc.store_scatter(ldstm, [tgt], dv, mask=mk)
                    if list_in:
                        ev = plsc.load_gather(ewm, [gsl])
                        plsc.store_scatter(ewm, [tgt], ev, mask=mk)

                def move_body(tt, _):
                    for q in range(UF):
                        move_one(tt * UF + q)
                    return 0
                lax.fori_loop(0, NT // UF, move_body, 0)
                return (OFF2 + Mc, W2)
            OFF_end, W_end = lax.fori_loop(0, NCH, chunk_body, (0, 0))
            flush(OFF_end, W_end)

            if list_out:
                Wfin = W_end + lax.bitwise_and(OFF_end + 15, -16)
                cstage[pl.ds(0, 16)] = jnp.full((L,), 0, jnp.int32) + Wfin
                pltpu.sync_copy(cstage, lcnt_hbm.at[pl.ds(wid * 16, 16)])

            def fin_body(t, _):
                v = acc[pl.ds(t * 16, 16)]
                acc[pl.ds(t * 16, 16)] = jnp.where(v == neg_inf, 0.0, v)
                return 0
            lax.fori_loop(0, RS * F // 16, fin_body, 0)
            pltpu.sync_copy(acc.at[pl.ds(0, RS * F)],
                            out_hbm.at[pl.ds(lo * F, RS * F)])

    mesh = plsc.VectorSubcoreMesh(core_axis_name="c", subcore_axis_name="s",
                                  num_cores=NC, num_subcores=NS)
    if list_out:
        out_type = (jax.ShapeDtypeStruct((NPAD * F,), jnp.float32),
                    jax.ShapeDtypeStruct((NW * ESEG,), jnp.int32),
                    jax.ShapeDtypeStruct((NW * ESEG,), jnp.int32),
                    jax.ShapeDtypeStruct((NW * ESEG,), jnp.float32),
                    jax.ShapeDtypeStruct((NW * 16,), jnp.int32))
    else:
        out_type = jax.ShapeDtypeStruct((NPAD * F,), jnp.float32)
    scratch = []
    if list_out:
        scratch.append(pltpu.VMEM((POSPAD,), jnp.float32))
    scratch.append(pltpu.VMEM((4 * ECL,), jnp.int32))
    if list_in:
        scratch.insert(1, pltpu.VMEM((2 * ECL,), jnp.float32))
    scratch += [
        pltpu.VMEM((MB,), jnp.int32),
        pltpu.VMEM((MB,), jnp.int32),
        pltpu.VMEM((MB,), jnp.float32),
        pltpu.VMEM((2 * NT,), jnp.int32),
        pltpu.VMEM((16,), jnp.int32),
        pltpu.VMEM((2 * GX, F), jnp.float32),
        pltpu.VMEM(((RS + 1) * F,), jnp.float32),
        pltpu.SemaphoreType.DMA((2, NSEM)),
        pltpu.SemaphoreType.DMA((2,)),
    ]
    return pl.kernel(
        body,
        out_type=out_type,
        mesh=mesh,
        compiler_params=pltpu.CompilerParams(needs_layout_passes=False),
        scratch_types=scratch,
    )


_segmax1 = _make_segmax(D, NPAD // NW, 1, 8192, 32, 8, 4000, 5,
                        list_out=True, list_in=False)
_segmax2 = _make_segmax(H, NPAD // (2 * NW), 2, 3072, 16, 4, 2048, 4,
                        list_out=False, list_in=True)


def _dense1_body(agg_ref, x_ref, wr_ref, wo_ref, b_ref, g_ref, be_ref, o_ref):
    z = jnp.dot(agg_ref[...], wr_ref[...], preferred_element_type=jnp.float32)
    z += jnp.dot(x_ref[...], wo_ref[...], preferred_element_type=jnp.float32)
    z += b_ref[...]
    mu = jnp.mean(z, axis=-1, keepdims=True)
    var = jnp.mean((z - mu) ** 2, axis=-1, keepdims=True)
    z = (z - mu) * lax.rsqrt(var + 1e-5) * g_ref[...] + be_ref[...]
    o_ref[...] = jnp.maximum(z, 0.0)


def _dense2_body(agg_ref, h_ref, wr_ref, wo_ref, b_ref, g_ref, be_ref,
                 wc_ref, bc_ref, o_ref):
    z = jnp.dot(agg_ref[...], wr_ref[...], preferred_element_type=jnp.float32)
    z += jnp.dot(h_ref[...], wo_ref[...], preferred_element_type=jnp.float32)
    z += b_ref[...]
    mu = jnp.mean(z, axis=-1, keepdims=True)
    var = jnp.mean((z - mu) ** 2, axis=-1, keepdims=True)
    z = (z - mu) * lax.rsqrt(var + 1e-5) * g_ref[...] + be_ref[...]
    z = jnp.maximum(z, 0.0)
    o_ref[...] = jnp.dot(z, wc_ref[...],
                         preferred_element_type=jnp.float32) + bc_ref[...]


_BN = 400


def _dense1(agg, x, wr, wo, b, g, be):
    return pl.pallas_call(
        _dense1_body,
        grid=(N // _BN,),
        in_specs=[
            pl.BlockSpec((_BN, D), lambda i: (i, 0)),
            pl.BlockSpec((_BN, D), lambda i: (i, 0)),
            pl.BlockSpec((D, H), lambda i: (0, 0)),
            pl.BlockSpec((D, H), lambda i: (0, 0)),
            pl.BlockSpec((1, H), lambda i: (0, 0)),
            pl.BlockSpec((1, H), lambda i: (0, 0)),
            pl.BlockSpec((1, H), lambda i: (0, 0)),
        ],
        out_specs=pl.BlockSpec((_BN, H), lambda i: (i, 0)),
        out_shape=jax.ShapeDtypeStruct((N, H), jnp.float32),
    )(agg, x, wr, wo, b, g, be)


def _dense2(agg, h, wr, wo, b, g, be, wc, bc):
    return pl.pallas_call(
        _dense2_body,
        grid=(N // _BN,),
        in_specs=[
            pl.BlockSpec((_BN, H), lambda i: (i, 0)),
            pl.BlockSpec((_BN, H), lambda i: (i, 0)),
            pl.BlockSpec((H, H), lambda i: (0, 0)),
            pl.BlockSpec((H, H), lambda i: (0, 0)),
            pl.BlockSpec((1, H), lambda i: (0, 0)),
            pl.BlockSpec((1, H), lambda i: (0, 0)),
            pl.BlockSpec((1, H), lambda i: (0, 0)),
            pl.BlockSpec((H, 128), lambda i: (0, 0)),
            pl.BlockSpec((1, 128), lambda i: (0, 0)),
        ],
        out_specs=pl.BlockSpec((_BN, 128), lambda i: (i, 0)),
        out_shape=jax.ShapeDtypeStruct((N, 128), jnp.float32),
    )(agg, h, wr, wo, b, g, be, wc, bc)


def kernel(x, edge_index, pos, W_rel1, b_rel1, W_root1, g1, be1,
           W_rel2, b_rel2, W_root2, g2, be2, W_cls, b_cls):
    src = edge_index[0]
    dst = edge_index[1]
    agg1f, lsrc, lldst, lew, lcnt = _segmax1(x, src, dst, pos)
    agg1 = agg1f.reshape(NPAD, D)[:N]
    h1 = _dense1(agg1, x, W_rel1, W_root1, b_rel1.reshape(1, H),
                 g1.reshape(1, H), be1.reshape(1, H))
    agg2 = _segmax2(h1, lsrc, lldst, lew, lcnt).reshape(NPAD, H)[:N]
    wc = jnp.zeros((H, 128), jnp.float32).at[:, :C].set(W_cls)
    bc = jnp.zeros((1, 128), jnp.float32).at[0, :C].set(b_cls)
    out = _dense2(agg2, h1, W_rel2, W_root2, b_rel2.reshape(1, H),
                  g2.reshape(1, H), be2.reshape(1, H), wc, bc)
    return out[:, :C]
